# baseline (speedup 1.0000x reference)
import functools

import jax
import jax.numpy as jnp
from jax import lax
from jax.experimental import pallas as pl
from jax.experimental.pallas import tpu as pltpu
from jax.experimental.pallas import tpu_sc as plsc

NC = 2
NS = 16
NW = NC * NS
L = 16


def _rsqrt16(x):
    i = lax.bitcast_convert_type(x, jnp.int32)
    i = jnp.int32(0x5F3759DF) - (i >> 1)
    y = lax.bitcast_convert_type(i, jnp.float32)
    xh = x * 0.5
    for _ in range(3):
        y = y * (1.5 - xh * y * y)
    return y


def _make_sc_kernel(B, S, V, D):
    assert B % NW == 0 and D % L == 0
    rows_per_w = B // NW
    n_groups = (S + L - 1) // L
    ids_pad = n_groups * L
    row_elems = S * D

    mesh = plsc.VectorSubcoreMesh(core_axis_name="c", subcore_axis_name="s",
                                  num_cores=NC, num_subcores=NS)
    out_t = jax.ShapeDtypeStruct((B * S * D,), jnp.float32)

    @functools.partial(
        pl.kernel,
        out_type=(out_t, out_t, out_t),
        mesh=mesh,
        compiler_params=pltpu.CompilerParams(needs_layout_passes=False),
        scratch_types=[
            pltpu.VMEM((V * D,), jnp.float32),
            pltpu.VMEM((D,), jnp.float32),
            pltpu.VMEM((D,), jnp.float32),
            pltpu.VMEM((ids_pad,), jnp.int32),
            pltpu.VMEM((row_elems,), jnp.float32),
            pltpu.VMEM((row_elems,), jnp.float32),
            pltpu.VMEM((row_elems,), jnp.float32),
            pltpu.VMEM((D, L), jnp.float32),
            pltpu.VMEM((L,), jnp.float32),
        ],
    )
    def sc_kernel(ids_hbm, tab_hbm, g_hbm, bt_hbm, h0_hbm, h1_hbm, h2_hbm,
                  tab_v, g_v, bt_v, ids_v, h0b, h1b, h2b, h1lin, red_v):
        wid = lax.axis_index("s") * NC + lax.axis_index("c")
        pltpu.sync_copy(tab_hbm, tab_v)
        pltpu.sync_copy(g_hbm, g_v)
        pltpu.sync_copy(bt_hbm, bt_v)

        lane = lax.iota(jnp.int32, L)
        zero16 = jnp.zeros((L,), jnp.float32)
        g_vecs = [g_v[pl.ds(j * L, L)] for j in range(D // L)]
        bt_vecs = [bt_v[pl.ds(j * L, L)] for j in range(D // L)]
        g_s = [g_vecs[d // L][d % L] for d in range(D)]
        bt_s = [bt_vecs[d // L][d % L] for d in range(D)]
        if ids_pad != S:
            ids_v[pl.ds(ids_pad - L, L)] = jnp.zeros((L,), jnp.int32)

        def row_body(r, _):
            b = wid * rows_per_w + r
            pltpu.sync_copy(ids_hbm.at[pl.ds(b * S, S)], ids_v.at[pl.ds(0, S)])
            tot = zero16
            for j in range(n_groups):
                tot = tot + ids_v[pl.ds(j * L, L)].astype(jnp.float32)
            for k in (8, 4, 2, 1):
                red_v[...] = tot
                tot = tot + plsc.load_gather(red_v, [lane ^ k])
            bias = tot * (0.05 / S)

            def grp(g, _):
                idv = ids_v[pl.ds(g * L, L)]
                base = idv * D
                tok = g * L + lane
                mask = tok < S
                obase = tok * D

                s0 = zero16
                s1 = zero16
                for d in range(D):
                    x = plsc.load_gather(tab_v, [base + d])
                    s0 = s0 + x
                    s1 = s1 + x * x
                mu = s0 * (1.0 / D)
                var = s1 * (1.0 / D) - mu * mu
                r0 = _rsqrt16(var + 1e-5)

                s2 = zero16
                s3 = zero16
                for d in range(D):
                    x = plsc.load_gather(tab_v, [base + d])
                    plsc.store_scatter(h0b, [obase + d], x + bias, mask=mask)
                    h1 = (x - mu) * r0 * g_s[d] + bt_s[d]
                    s2 = s2 + h1
                    s3 = s3 + h1 * h1
                    h1lin[d] = h1
                    plsc.store_scatter(h1b, [obase + d], h1, mask=mask)
                mu1 = s2 * (1.0 / D)
                var1 = s3 * (1.0 / D) - mu1 * mu1
                r1 = _rsqrt16(var1 + 1e-5)

                for d in range(D):
                    h1 = h1lin[d]
                    h2 = (h1 - mu1) * r1 * g_s[d] + bt_s[d]
                    plsc.store_scatter(h2b, [obase + d], h2, mask=mask)
                return _

            lax.fori_loop(0, n_groups, grp, None)
            dst = pl.ds(b * row_elems, row_elems)
            pltpu.sync_copy(h0b, h0_hbm.at[dst])
            pltpu.sync_copy(h1b, h1_hbm.at[dst])
            pltpu.sync_copy(h2b, h2_hbm.at[dst])
            return _

        lax.fori_loop(0, rows_per_w, row_body, None)

    return sc_kernel


def kernel(input_ids, table, gamma, beta):
    B, S = input_ids.shape
    V, D = table.shape
    ids_flat = input_ids.reshape(-1).astype(jnp.int32)
    tab_flat = table.reshape(-1).astype(jnp.float32)
    sc = _make_sc_kernel(B, S, V, D)
    h0, h1, h2 = sc(ids_flat, tab_flat,
                    gamma.astype(jnp.float32), beta.astype(jnp.float32))
    shp = (B, S, D)
    return h0.reshape(shp), h1.reshape(shp), h2.reshape(shp)


# speedup vs baseline: 1.3366x; 1.3366x over previous
import functools

import jax
import jax.numpy as jnp
from jax import lax
from jax.experimental import pallas as pl
from jax.experimental.pallas import tpu as pltpu
from jax.experimental.pallas import tpu_sc as plsc

NC = 2
NS = 16
NW = NC * NS
L = 16


def _rsqrt16(x):
    i = lax.bitcast_convert_type(x, jnp.int32)
    i = jnp.int32(0x5F3759DF) - (i >> 1)
    y = lax.bitcast_convert_type(i, jnp.float32)
    xh = x * 0.5
    for _ in range(3):
        y = y * (1.5 - xh * y * y)
    return y


def _make_sc_kernel(B, S, V, D):
    assert B % NW == 0 and D % L == 0 and V % L == 0
    rows_per_w = B // NW
    CH = 112
    NCHK = -(-S // CH)
    SP = NCHK * CH
    assert S % 8 == 0 and CH % 8 == 0

    mesh = plsc.VectorSubcoreMesh(core_axis_name="c", subcore_axis_name="s",
                                  num_cores=NC, num_subcores=NS)
    out_t = jax.ShapeDtypeStruct((B * S, D), jnp.float32)
    tbl_t = jax.ShapeDtypeStruct((NW * V, D), jnp.float32)

    @functools.partial(
        pl.kernel,
        out_type=(out_t, out_t, out_t, tbl_t, tbl_t),
        mesh=mesh,
        compiler_params=pltpu.CompilerParams(needs_layout_passes=False,
                                             use_tc_tiling_on_sc=False),
        scratch_types=[
            pltpu.VMEM((V, D), jnp.float32),
            pltpu.VMEM((D,), jnp.float32),
            pltpu.VMEM((D,), jnp.float32),
            pltpu.VMEM((V, D), jnp.float32),
            pltpu.VMEM((V, D), jnp.float32),
            pltpu.VMEM((D, L), jnp.float32),
            pltpu.VMEM((L,), jnp.float32),
            pltpu.VMEM((NCHK, CH), jnp.int32),
            pltpu.VMEM((NCHK, CH), jnp.int32),
            pltpu.VMEM((SP, D), jnp.float32),
            pltpu.VMEM((SP, D), jnp.float32),
            pltpu.VMEM((SP, D), jnp.float32),
            pltpu.SemaphoreType.DMA,
        ],
    )
    def sc_kernel(ids_hbm, tab_hbm, g_hbm, bt_hbm,
                  h0_hbm, h1_hbm, h2_hbm, t1_hbm, t2_hbm,
                  tab_v, g_v, bt_v, t1_v, t2_v, h1lin, red_v,
                  idxp, idx1, h0b, h1b, h2b, sem):
        wid = lax.axis_index("s") * NC + lax.axis_index("c")
        pltpu.sync_copy(tab_hbm, tab_v)
        pltpu.sync_copy(g_hbm, g_v)
        pltpu.sync_copy(bt_hbm, bt_v)

        lane = lax.iota(jnp.int32, L)
        zero16 = jnp.zeros((L,), jnp.float32)
        g_vecs = [g_v[pl.ds(j * L, L)] for j in range(D // L)]
        bt_vecs = [bt_v[pl.ds(j * L, L)] for j in range(D // L)]
        g_s = [g_vecs[d // L][d % L] for d in range(D)]
        bt_s = [bt_vecs[d // L][d % L] for d in range(D)]

        for g2 in range(V // L):
            idv = g2 * L + lane
            s0 = zero16
            s1 = zero16
            for d in range(D):
                x = plsc.load_gather(tab_v, [idv, jnp.full((L,), d, jnp.int32)])
                s0 = s0 + x
                s1 = s1 + x * x
            mu = s0 * (1.0 / D)
            var = s1 * (1.0 / D) - mu * mu
            r0 = _rsqrt16(var + 1e-5)
            s2 = zero16
            s3 = zero16
            for d in range(D):
                x = plsc.load_gather(tab_v, [idv, jnp.full((L,), d, jnp.int32)])
                h1 = (x - mu) * r0 * g_s[d] + bt_s[d]
                s2 = s2 + h1
                s3 = s3 + h1 * h1
                h1lin[d] = h1
                plsc.store_scatter(t1_v, [idv, jnp.full((L,), d, jnp.int32)], h1)
            mu1 = s2 * (1.0 / D)
            var1 = s3 * (1.0 / D) - mu1 * mu1
            r1 = _rsqrt16(var1 + 1e-5)
            for d in range(D):
                h2 = (h1lin[d] - mu1) * r1 * g_s[d] + bt_s[d]
                plsc.store_scatter(t2_v, [idv, jnp.full((L,), d, jnp.int32)], h2)
        dstv = pl.ds(wid * V, V)
        pltpu.sync_copy(t1_v, t1_hbm.at[dstv])
        pltpu.sync_copy(t2_v, t2_hbm.at[dstv])

        tail = S - (NCHK - 1) * CH
        pad0 = (tail // L) * L
        for off in range(pad0, CH, L):
            idxp[NCHK - 1, pl.ds(off, L)] = jnp.zeros((L,), jnp.int32)
        wV = wid * V

        def row_body(r, _):
            b = wid * rows_per_w + r
            for c in range(NCHK - 1):
                pltpu.sync_copy(ids_hbm.at[pl.ds(b * S + c * CH, CH)],
                                idxp.at[c])
            pltpu.sync_copy(ids_hbm.at[pl.ds(b * S + (NCHK - 1) * CH, tail)],
                            idxp.at[NCHK - 1].at[pl.ds(0, tail)])
            tot = zero16
            for c in range(NCHK):
                for k in range(CH // L):
                    v = idxp[c, pl.ds(k * L, L)]
                    tot = tot + v.astype(jnp.float32)
                    idx1[c, pl.ds(k * L, L)] = v + wV
            for k in (8, 4, 2, 1):
                red_v[...] = tot
                tot = tot + plsc.load_gather(red_v, [lane ^ k])
            bias = tot * (0.05 / S)
            for c in range(NCHK):
                dst = pl.ds(c * CH, CH)
                pltpu.async_copy(tab_hbm.at[idxp.at[c]], h0b.at[dst], sem).wait()
                pltpu.async_copy(t1_hbm.at[idx1.at[c]], h1b.at[dst], sem).wait()
                pltpu.async_copy(t2_hbm.at[idx1.at[c]], h2b.at[dst], sem).wait()

            def tkn(t, _):
                for k in range(D // L):
                    sl = pl.ds(k * L, L)
                    h0b[t, sl] = h0b[t, sl] + bias
                return _

            lax.fori_loop(0, S, tkn, None)
            src = pl.ds(0, S)
            dsto = pl.ds(b * S, S)
            pltpu.sync_copy(h0b.at[src], h0_hbm.at[dsto])
            pltpu.sync_copy(h1b.at[src], h1_hbm.at[dsto])
            pltpu.sync_copy(h2b.at[src], h2_hbm.at[dsto])
            return _

        lax.fori_loop(0, rows_per_w, row_body, None)

    return sc_kernel


def kernel(input_ids, table, gamma, beta):
    B, S = input_ids.shape
    V, D = table.shape
    ids_flat = input_ids.reshape(-1).astype(jnp.int32)
    sc = _make_sc_kernel(B, S, V, D)
    h0, h1, h2, _, _ = sc(ids_flat, table.astype(jnp.float32),
                          gamma.astype(jnp.float32), beta.astype(jnp.float32))
    shp = (B, S, D)
    return h0.reshape(shp), h1.reshape(shp), h2.reshape(shp)


# speedup vs baseline: 1.3495x; 1.0096x over previous
import functools

import jax
import jax.numpy as jnp
from jax import lax
from jax.experimental import pallas as pl
from jax.experimental.pallas import tpu as pltpu
from jax.experimental.pallas import tpu_sc as plsc

NC = 2
NS = 16
NW = NC * NS
L = 16


def _rsqrt16(x):
    i = lax.bitcast_convert_type(x, jnp.int32)
    i = jnp.int32(0x5F3759DF) - (i >> 1)
    y = lax.bitcast_convert_type(i, jnp.float32)
    xh = x * 0.5
    for _ in range(3):
        y = y * (1.5 - xh * y * y)
    return y


def _make_sc_kernel(B, S, V, D):
    assert B % NW == 0 and D % L == 0 and V % L == 0
    rows_per_w = B // NW
    assert rows_per_w >= 3
    CH = 112
    NCHK = -(-S // CH)
    SP = NCHK * CH
    assert S % 8 == 0 and CH % 8 == 0
    nid = rows_per_w * S
    stage_n = -(-(nid + NCHK * CH) // L) * L

    slices = []
    for c in range(NCHK):
        for k in range(CH // L):
            pos = c * CH + k * L
            nv = max(0, min(L, S - pos))
            slices.append((c, k, nv))

    mesh = plsc.VectorSubcoreMesh(core_axis_name="c", subcore_axis_name="s",
                                  num_cores=NC, num_subcores=NS)
    out_t = jax.ShapeDtypeStruct((B, S, D), jnp.float32)
    tbl_t = jax.ShapeDtypeStruct((NW * V, D), jnp.float32)

    @functools.partial(
        pl.kernel,
        out_type=(out_t, out_t, out_t, tbl_t, tbl_t),
        mesh=mesh,
        compiler_params=pltpu.CompilerParams(needs_layout_passes=False,
                                             use_tc_tiling_on_sc=False),
        scratch_types=[
            pltpu.VMEM((V, D), jnp.float32),
            pltpu.VMEM((D,), jnp.float32),
            pltpu.VMEM((D,), jnp.float32),
            pltpu.VMEM((V, D), jnp.float32),
            pltpu.VMEM((V, D), jnp.float32),
            pltpu.VMEM((D, L), jnp.float32),
            pltpu.VMEM((L,), jnp.float32),
            pltpu.VMEM((stage_n,), jnp.int32),
            pltpu.VMEM((rows_per_w * NCHK, CH), jnp.int32),
            pltpu.VMEM((rows_per_w * NCHK, CH), jnp.int32),
            pltpu.VMEM((SP, D), jnp.float32),
            pltpu.VMEM((SP, D), jnp.float32),
            pltpu.VMEM((SP, D), jnp.float32),
            pltpu.VMEM((SP, D), jnp.float32),
            pltpu.VMEM((SP, D), jnp.float32),
            pltpu.VMEM((SP, D), jnp.float32),
            pltpu.SemaphoreType.DMA,
            pltpu.SemaphoreType.DMA,
            pltpu.SemaphoreType.DMA,
            pltpu.SemaphoreType.DMA,
        ],
    )
    def sc_kernel(ids_hbm, tab_hbm, g_hbm, bt_hbm,
                  h0_hbm, h1_hbm, h2_hbm, t1_hbm, t2_hbm,
                  tab_v, g_v, bt_v, t1_v, t2_v, h1lin, red_v,
                  stage_v, idxp, idx1,
                  h0b0, h1b0, h2b0, h0b1, h1b1, h2b1,
                  sem_g0, sem_g1, sem_o0, sem_o1):
        wid = lax.axis_index("s") * NC + lax.axis_index("c")
        h0b = [h0b0, h0b1]
        h1b = [h1b0, h1b1]
        h2b = [h2b0, h2b1]
        sem_g = [sem_g0, sem_g1]
        sem_o = [sem_o0, sem_o1]
        pltpu.sync_copy(tab_hbm, tab_v)
        pltpu.sync_copy(g_hbm, g_v)
        pltpu.sync_copy(bt_hbm, bt_v)

        lane = lax.iota(jnp.int32, L)
        zero16 = jnp.zeros((L,), jnp.float32)
        g_vecs = [g_v[pl.ds(j * L, L)] for j in range(D // L)]
        bt_vecs = [bt_v[pl.ds(j * L, L)] for j in range(D // L)]
        g_s = [g_vecs[d // L][d % L] for d in range(D)]
        bt_s = [bt_vecs[d // L][d % L] for d in range(D)]

        def splat_i(v):
            return jnp.broadcast_to(v, (L,)).astype(jnp.int32)

        for g2 in range(V // L):
            idv = g2 * L + lane
            s0 = zero16
            s1 = zero16
            for d in range(D):
                x = plsc.load_gather(tab_v, [idv, splat_i(d)])
                s0 = s0 + x
                s1 = s1 + x * x
            mu = s0 * (1.0 / D)
            var = s1 * (1.0 / D) - mu * mu
            r0 = _rsqrt16(var + 1e-5)
            s2 = zero16
            s3 = zero16
            for d in range(D):
                x = plsc.load_gather(tab_v, [idv, splat_i(d)])
                h1 = (x - mu) * r0 * g_s[d] + bt_s[d]
                s2 = s2 + h1
                s3 = s3 + h1 * h1
                h1lin[d] = h1
                plsc.store_scatter(t1_v, [idv, splat_i(d)], h1)
            mu1 = s2 * (1.0 / D)
            var1 = s3 * (1.0 / D) - mu1 * mu1
            r1 = _rsqrt16(var1 + 1e-5)
            for d in range(D):
                h2 = (h1lin[d] - mu1) * r1 * g_s[d] + bt_s[d]
                plsc.store_scatter(t2_v, [idv, splat_i(d)], h2)
        dstv = pl.ds(wid * V, V)
        pltpu.sync_copy(t1_v, t1_hbm.at[dstv])
        pltpu.sync_copy(t2_v, t2_hbm.at[dstv])

        pltpu.sync_copy(ids_hbm.at[pl.ds(wid * nid, nid)],
                        stage_v.at[pl.ds(0, nid)])
        wV = wid * V

        def build_row(r, _):
            for (c, k, nv) in slices:
                v = stage_v[pl.ds(r * S + c * CH + k * L, L)]
                if nv < L:
                    v = jnp.where(lane < nv, v, 0)
                idxp[r * NCHK + c, pl.ds(k * L, L)] = v
                idx1[r * NCHK + c, pl.ds(k * L, L)] = v + wV
            return _

        lax.fori_loop(0, rows_per_w, build_row, None)

        def fire_gathers(r, p):
            for c in range(NCHK):
                i2 = r * NCHK + c
                dst = pl.ds(c * CH, CH)
                pltpu.async_copy(tab_hbm.at[idxp.at[i2]], h0b[p].at[dst],
                                 sem_g[p])
                pltpu.async_copy(t1_hbm.at[idx1.at[i2]], h1b[p].at[dst],
                                 sem_g[p])
                pltpu.async_copy(t2_hbm.at[idx1.at[i2]], h2b[p].at[dst],
                                 sem_g[p])

        def wait_gathers(r, p):
            for c in range(NCHK):
                i2 = r * NCHK + c
                dst = pl.ds(c * CH, CH)
                pltpu.make_async_copy(tab_hbm.at[idxp.at[i2]],
                                      h0b[p].at[dst], sem_g[p]).wait()
                pltpu.make_async_copy(t1_hbm.at[idx1.at[i2]],
                                      h1b[p].at[dst], sem_g[p]).wait()
                pltpu.make_async_copy(t2_hbm.at[idx1.at[i2]],
                                      h2b[p].at[dst], sem_g[p]).wait()

        def bias_pass(r, p):
            tot = zero16
            for (c, k, nv) in slices:
                if nv > 0:
                    tot = tot + idxp[r * NCHK + c,
                                     pl.ds(k * L, L)].astype(jnp.float32)
            for kk in (8, 4, 2, 1):
                red_v[...] = tot
                tot = tot + plsc.load_gather(red_v, [lane ^ kk])
            bias = tot * (0.05 / S)

            def tkn(t, _):
                for k in range(D // L):
                    sl = pl.ds(k * L, L)
                    h0b[p][t, sl] = h0b[p][t, sl] + bias
                return _

            lax.fori_loop(0, S, tkn, None)

        def fire_outs(r, p):
            b = wid * rows_per_w + r
            src = pl.ds(0, S)
            pltpu.async_copy(h0b[p].at[src], h0_hbm.at[b], sem_o[p])
            pltpu.async_copy(h1b[p].at[src], h1_hbm.at[b], sem_o[p])
            pltpu.async_copy(h2b[p].at[src], h2_hbm.at[b], sem_o[p])

        def wait_outs(r, p):
            b = wid * rows_per_w + r
            src = pl.ds(0, S)
            pltpu.make_async_copy(h0b[p].at[src], h0_hbm.at[b],
                                  sem_o[p]).wait()
            pltpu.make_async_copy(h1b[p].at[src], h1_hbm.at[b],
                                  sem_o[p]).wait()
            pltpu.make_async_copy(h2b[p].at[src], h2_hbm.at[b],
                                  sem_o[p]).wait()

        def step(r, p):
            wait_gathers(r, p)
            bias_pass(r, p)
            fire_outs(r, p)
            wait_outs(r - 1, p ^ 1)
            fire_gathers(r + 1, p ^ 1)

        fire_gathers(0, 0)
        fire_gathers(1, 1)
        wait_gathers(0, 0)
        bias_pass(0, 0)
        fire_outs(0, 0)

        assert rows_per_w % 2 == 0

        def row_pair(i, _):
            step(2 * i + 1, 1)
            step(2 * i + 2, 0)
            return _

        lax.fori_loop(0, (rows_per_w - 2) // 2, row_pair, None)

        rl = rows_per_w - 1
        wait_gathers(rl, 1)
        bias_pass(rl, 1)
        fire_outs(rl, 1)
        wait_outs(rl - 1, 0)
        wait_outs(rl, 1)

    return sc_kernel


def kernel(input_ids, table, gamma, beta):
    B, S = input_ids.shape
    V, D = table.shape
    ids_flat = input_ids.reshape(-1).astype(jnp.int32)
    sc = _make_sc_kernel(B, S, V, D)
    h0, h1, h2, _, _ = sc(ids_flat, table.astype(jnp.float32),
                          gamma.astype(jnp.float32), beta.astype(jnp.float32))
    return h0, h1, h2


# speedup vs baseline: 2.9563x; 2.1907x over previous
import functools

import jax
import jax.numpy as jnp
from jax import lax
from jax.experimental import pallas as pl
from jax.experimental.pallas import tpu as pltpu
from jax.experimental.pallas import tpu_sc as plsc

NC = 2
NS = 16
NW = NC * NS
L = 16


def _rsqrt16(x):
    i = lax.bitcast_convert_type(x, jnp.int32)
    i = jnp.int32(0x5F3759DF) - (i >> 1)
    y = lax.bitcast_convert_type(i, jnp.float32)
    xh = x * 0.5
    for _ in range(3):
        y = y * (1.5 - xh * y * y)
    return y


def _make_sc_kernel(B, S, V, D):
    assert B % NW == 0 and D % L == 0 and V % L == 0
    rows_per_w = B // NW
    assert rows_per_w >= 4 and rows_per_w % 2 == 0
    VP = V + 1
    n_groups = -(-S // L)
    nid = rows_per_w * S
    stage_n = -(-(nid + L) // L) * L

    mesh = plsc.VectorSubcoreMesh(core_axis_name="c", subcore_axis_name="s",
                                  num_cores=NC, num_subcores=NS)
    out_t = jax.ShapeDtypeStruct((B, S, D), jnp.float32)

    @functools.partial(
        pl.kernel,
        out_type=(out_t, out_t, out_t),
        mesh=mesh,
        compiler_params=pltpu.CompilerParams(needs_layout_passes=False,
                                             use_tc_tiling_on_sc=False),
        scratch_types=[
            pltpu.VMEM((V, D), jnp.float32),
            pltpu.VMEM((D,), jnp.float32),
            pltpu.VMEM((D,), jnp.float32),
            pltpu.VMEM((D * VP,), jnp.float32),
            pltpu.VMEM((D * VP,), jnp.float32),
            pltpu.VMEM((D * VP,), jnp.float32),
            pltpu.VMEM((D, L), jnp.float32),
            pltpu.VMEM((L,), jnp.float32),
            pltpu.VMEM((stage_n,), jnp.int32),
            pltpu.VMEM((S, D), jnp.float32),
            pltpu.VMEM((S, D), jnp.float32),
            pltpu.VMEM((S, D), jnp.float32),
            pltpu.VMEM((S, D), jnp.float32),
            pltpu.VMEM((S, D), jnp.float32),
            pltpu.VMEM((S, D), jnp.float32),
            pltpu.SemaphoreType.DMA,
            pltpu.SemaphoreType.DMA,
        ],
    )
    def sc_kernel(ids_hbm, tab_hbm, g_hbm, bt_hbm,
                  h0_hbm, h1_hbm, h2_hbm,
                  tab_v, g_v, bt_v, t0p, t1p, t2p, h1lin, red_v, stage_v,
                  h0b0, h1b0, h2b0, h0b1, h1b1, h2b1, sem_o0, sem_o1):
        wid = lax.axis_index("s") * NC + lax.axis_index("c")
        h0b = [h0b0, h0b1]
        h1b = [h1b0, h1b1]
        h2b = [h2b0, h2b1]
        sem_o = [sem_o0, sem_o1]
        pltpu.sync_copy(tab_hbm, tab_v)
        pltpu.sync_copy(g_hbm, g_v)
        pltpu.sync_copy(bt_hbm, bt_v)
        pltpu.sync_copy(ids_hbm.at[pl.ds(wid * nid, nid)],
                        stage_v.at[pl.ds(0, nid)])

        lane = lax.iota(jnp.int32, L)
        zero16 = jnp.zeros((L,), jnp.float32)
        g_vecs = [g_v[pl.ds(j * L, L)] for j in range(D // L)]
        bt_vecs = [bt_v[pl.ds(j * L, L)] for j in range(D // L)]
        g_s = [g_vecs[d // L][d % L] for d in range(D)]
        bt_s = [bt_vecs[d // L][d % L] for d in range(D)]

        def splat_i(v):
            return jnp.broadcast_to(v, (L,)).astype(jnp.int32)

        for g2 in range(V // L):
            idv = g2 * L + lane
            s0 = zero16
            s1 = zero16
            for d in range(D):
                x = plsc.load_gather(tab_v, [idv, splat_i(d)])
                plsc.store_scatter(t0p, [idv + d * VP], x)
                s0 = s0 + x
                s1 = s1 + x * x
            mu = s0 * (1.0 / D)
            var = s1 * (1.0 / D) - mu * mu
            r0 = _rsqrt16(var + 1e-5)
            s2 = zero16
            s3 = zero16
            for d in range(D):
                x = plsc.load_gather(tab_v, [idv, splat_i(d)])
                h1 = (x - mu) * r0 * g_s[d] + bt_s[d]
                s2 = s2 + h1
                s3 = s3 + h1 * h1
                h1lin[d] = h1
                plsc.store_scatter(t1p, [idv + d * VP], h1)
            mu1 = s2 * (1.0 / D)
            var1 = s3 * (1.0 / D) - mu1 * mu1
            r1 = _rsqrt16(var1 + 1e-5)
            for d in range(D):
                h2 = (h1lin[d] - mu1) * r1 * g_s[d] + bt_s[d]
                plsc.store_scatter(t2p, [idv + d * VP], h2)

        perms = [(lane + j) & (L - 1) for j in range(L)]

        def compute_row(r, p):
            tot = zero16
            for g in range(n_groups):
                v = stage_v[pl.ds(r * S + g * L, L)]
                nv = min(L, S - g * L)
                if nv < L:
                    v = jnp.where(lane < nv, v, 0)
                tot = tot + v.astype(jnp.float32)
            for kk in (8, 4, 2, 1):
                red_v[...] = tot
                tot = tot + plsc.load_gather(red_v, [lane ^ kk])
            bias = tot * (0.05 / S)

            def grp(g, _):
                ids_vec = stage_v[pl.ds(r * S + g * L, L)]
                tok = g * L + lane
                mask = tok < S
                ids_vec = jnp.where(mask, ids_vec, 0)
                for j in range(L):
                    dperm = perms[j]
                    for d0 in range(0, D, L):
                        dvec = d0 + dperm
                        tidx = ids_vec + dvec * VP
                        x0 = plsc.load_gather(t0p, [tidx])
                        x1 = plsc.load_gather(t1p, [tidx])
                        x2 = plsc.load_gather(t2p, [tidx])
                        plsc.store_scatter(h0b[p], [tok, dvec], x0 + bias,
                                           mask=mask)
                        plsc.store_scatter(h1b[p], [tok, dvec], x1, mask=mask)
                        plsc.store_scatter(h2b[p], [tok, dvec], x2, mask=mask)
                return _

            lax.fori_loop(0, n_groups, grp, None)

        def fire_outs(r, p):
            b = wid * rows_per_w + r
            pltpu.async_copy(h0b[p], h0_hbm.at[b], sem_o[p])
            pltpu.async_copy(h1b[p], h1_hbm.at[b], sem_o[p])
            pltpu.async_copy(h2b[p], h2_hbm.at[b], sem_o[p])

        def wait_outs(r, p):
            b = wid * rows_per_w + r
            pltpu.make_async_copy(h0b[p], h0_hbm.at[b], sem_o[p]).wait()
            pltpu.make_async_copy(h1b[p], h1_hbm.at[b], sem_o[p]).wait()
            pltpu.make_async_copy(h2b[p], h2_hbm.at[b], sem_o[p]).wait()

        compute_row(0, 0)
        fire_outs(0, 0)
        compute_row(1, 1)
        fire_outs(1, 1)

        def step(r, p):
            wait_outs(r - 2, p)
            compute_row(r, p)
            fire_outs(r, p)

        def row_pair(i, _):
            step(2 * i + 2, 0)
            step(2 * i + 3, 1)
            return _

        lax.fori_loop(0, (rows_per_w - 2) // 2, row_pair, None)

        wait_outs(rows_per_w - 2, 0)
        wait_outs(rows_per_w - 1, 1)

    return sc_kernel


def kernel(input_ids, table, gamma, beta):
    B, S = input_ids.shape
    V, D = table.shape
    ids_flat = input_ids.reshape(-1).astype(jnp.int32)
    sc = _make_sc_kernel(B, S, V, D)
    h0, h1, h2 = sc(ids_flat, table.astype(jnp.float32),
                    gamma.astype(jnp.float32), beta.astype(jnp.float32))
    return h0, h1, h2


# speedup vs baseline: 3.6119x; 1.2217x over previous
import functools

import jax
import jax.numpy as jnp
from jax import lax
from jax.experimental import pallas as pl
from jax.experimental.pallas import tpu as pltpu
from jax.experimental.pallas import tpu_sc as plsc

NC = 2
NS = 16
NW = NC * NS
L = 16


def _rsqrt16(x):
    i = lax.bitcast_convert_type(x, jnp.int32)
    i = jnp.int32(0x5F3759DF) - (i >> 1)
    y = lax.bitcast_convert_type(i, jnp.float32)
    xh = x * 0.5
    for _ in range(3):
        y = y * (1.5 - xh * y * y)
    return y


def _make_sc_kernel(B, S, V, D):
    assert B % NW == 0 and D % L == 0 and V % L == 0
    rows_per_w = B // NW
    assert rows_per_w >= 4 and rows_per_w % 2 == 0
    VP = D + 8
    n_groups = -(-S // L)
    full_groups = S // L
    nid = rows_per_w * S
    stage_n = -(-(nid + L) // L) * L

    mesh = plsc.VectorSubcoreMesh(core_axis_name="c", subcore_axis_name="s",
                                  num_cores=NC, num_subcores=NS)
    out_t = jax.ShapeDtypeStruct((B, S, D), jnp.float32)

    @functools.partial(
        pl.kernel,
        out_type=(out_t, out_t, out_t),
        mesh=mesh,
        compiler_params=pltpu.CompilerParams(needs_layout_passes=False,
                                             use_tc_tiling_on_sc=False),
        scratch_types=[
            pltpu.VMEM((V, D), jnp.float32),
            pltpu.VMEM((D,), jnp.float32),
            pltpu.VMEM((D,), jnp.float32),
            pltpu.VMEM((V * VP,), jnp.float32),
            pltpu.VMEM((V * VP,), jnp.float32),
            pltpu.VMEM((V * VP,), jnp.float32),
            pltpu.VMEM((D, L), jnp.float32),
            pltpu.VMEM((L,), jnp.float32),
            pltpu.VMEM((stage_n,), jnp.int32),
            pltpu.VMEM((S, D), jnp.float32),
            pltpu.VMEM((S, D), jnp.float32),
            pltpu.VMEM((S, D), jnp.float32),
            pltpu.VMEM((S, D), jnp.float32),
            pltpu.VMEM((S, D), jnp.float32),
            pltpu.VMEM((S, D), jnp.float32),
            pltpu.SemaphoreType.DMA,
            pltpu.SemaphoreType.DMA,
        ],
    )
    def sc_kernel(ids_hbm, tab_hbm, g_hbm, bt_hbm,
                  h0_hbm, h1_hbm, h2_hbm,
                  tab_v, g_v, bt_v, t0p, t1p, t2p, h1lin, red_v, stage_v,
                  h0b0, h1b0, h2b0, h0b1, h1b1, h2b1, sem_o0, sem_o1):
        wid = lax.axis_index("s") * NC + lax.axis_index("c")
        h0b = [h0b0, h0b1]
        h1b = [h1b0, h1b1]
        h2b = [h2b0, h2b1]
        sem_o = [sem_o0, sem_o1]
        pltpu.sync_copy(tab_hbm, tab_v)
        pltpu.sync_copy(g_hbm, g_v)
        pltpu.sync_copy(bt_hbm, bt_v)
        pltpu.sync_copy(ids_hbm.at[pl.ds(wid * nid, nid)],
                        stage_v.at[pl.ds(0, nid)])

        lane = lax.iota(jnp.int32, L)
        zero16 = jnp.zeros((L,), jnp.float32)
        g_vecs = [g_v[pl.ds(j * L, L)] for j in range(D // L)]
        bt_vecs = [bt_v[pl.ds(j * L, L)] for j in range(D // L)]
        g_s = [g_vecs[d // L][d % L] for d in range(D)]
        bt_s = [bt_vecs[d // L][d % L] for d in range(D)]

        def splat_i(v):
            return jnp.broadcast_to(v, (L,)).astype(jnp.int32)

        for g2 in range(V // L):
            idv = g2 * L + lane
            s0 = zero16
            s1 = zero16
            for d in range(D):
                x = plsc.load_gather(tab_v, [idv, splat_i(d)])
                plsc.store_scatter(t0p, [idv * VP + d], x)
                s0 = s0 + x
                s1 = s1 + x * x
            mu = s0 * (1.0 / D)
            var = s1 * (1.0 / D) - mu * mu
            r0 = _rsqrt16(var + 1e-5)
            s2 = zero16
            s3 = zero16
            for d in range(D):
                x = plsc.load_gather(tab_v, [idv, splat_i(d)])
                h1 = (x - mu) * r0 * g_s[d] + bt_s[d]
                s2 = s2 + h1
                s3 = s3 + h1 * h1
                h1lin[d] = h1
                plsc.store_scatter(t1p, [idv * VP + d], h1)
            mu1 = s2 * (1.0 / D)
            var1 = s3 * (1.0 / D) - mu1 * mu1
            r1 = _rsqrt16(var1 + 1e-5)
            for d in range(D):
                h2 = (h1lin[d] - mu1) * r1 * g_s[d] + bt_s[d]
                plsc.store_scatter(t2p, [idv * VP + d], h2)

        def compute_row(r, p):
            tot = zero16
            for g in range(n_groups):
                v = stage_v[pl.ds(r * S + g * L, L)]
                nv = min(L, S - g * L)
                if nv < L:
                    v = jnp.where(lane < nv, v, 0)
                tot = tot + v.astype(jnp.float32)
            for kk in (8, 4, 2, 1):
                red_v[...] = tot
                tot = tot + plsc.load_gather(red_v, [lane ^ kk])
            bias = tot * (0.05 / S)

            def emit_token(tok, id_s):
                base = id_s * VP
                for d0 in range(0, D, L):
                    sl = pl.ds(d0, L)
                    x0 = t0p[pl.ds(base + d0, L)]
                    x1 = t1p[pl.ds(base + d0, L)]
                    x2 = t2p[pl.ds(base + d0, L)]
                    h0b[p][tok, sl] = x0 + bias
                    h1b[p][tok, sl] = x1
                    h2b[p][tok, sl] = x2

            def grp(g, _):
                ids_vec = stage_v[pl.ds(r * S + g * L, L)]
                for i in range(L):
                    emit_token(g * L + i, ids_vec[i])
                return _

            lax.fori_loop(0, full_groups, grp, None)
            if full_groups < n_groups:
                gt = full_groups
                ids_vec = stage_v[pl.ds(r * S + gt * L, L)]
                for i in range(S - gt * L):
                    emit_token(gt * L + i, ids_vec[i])

        def fire_outs(r, p):
            b = wid * rows_per_w + r
            pltpu.async_copy(h0b[p], h0_hbm.at[b], sem_o[p])
            pltpu.async_copy(h1b[p], h1_hbm.at[b], sem_o[p])
            pltpu.async_copy(h2b[p], h2_hbm.at[b], sem_o[p])

        def wait_outs(r, p):
            b = wid * rows_per_w + r
            pltpu.make_async_copy(h0b[p], h0_hbm.at[b], sem_o[p]).wait()
            pltpu.make_async_copy(h1b[p], h1_hbm.at[b], sem_o[p]).wait()
            pltpu.make_async_copy(h2b[p], h2_hbm.at[b], sem_o[p]).wait()

        compute_row(0, 0)
        fire_outs(0, 0)
        compute_row(1, 1)
        fire_outs(1, 1)

        def step(r, p):
            wait_outs(r - 2, p)
            compute_row(r, p)
            fire_outs(r, p)

        def row_pair(i, _):
            step(2 * i + 2, 0)
            step(2 * i + 3, 1)
            return _

        lax.fori_loop(0, (rows_per_w - 2) // 2, row_pair, None)

        wait_outs(rows_per_w - 2, 0)
        wait_outs(rows_per_w - 1, 1)

    return sc_kernel


def kernel(input_ids, table, gamma, beta):
    B, S = input_ids.shape
    V, D = table.shape
    ids_flat = input_ids.reshape(-1).astype(jnp.int32)
    sc = _make_sc_kernel(B, S, V, D)
    h0, h1, h2 = sc(ids_flat, table.astype(jnp.float32),
                    gamma.astype(jnp.float32), beta.astype(jnp.float32))
    return h0, h1, h2
